# baseline (speedup 1.0000x reference)
import dataclasses
import functools

import jax
import jax.numpy as jnp
from jax import lax
from jax.experimental import pallas as pl
from jax.experimental.pallas import tpu as pltpu
from jax.experimental.pallas import tpu_sc as plsc

N_NODES = 10000
N_EDGES = 320000
D = 128

E_BLK = 128
N_SUBCORES = 16
N_WORKERS = 2 * N_SUBCORES
NBUF = 3
BLOCKS_PER_TILE = 81
N_BLOCKS = N_WORKERS * BLOCKS_PER_TILE
E_PAD = N_BLOCKS * E_BLK
ROWS_MAIN = 624
ROWS_TAIL = N_NODES - N_SUBCORES * ROWS_MAIN


def _mm_body(x_ref, w_ref, h_ref):
    h_ref[...] = jnp.dot(x_ref[...], w_ref[...],
                         preferred_element_type=jnp.float32)


def _matmul(x, W):
    grid = 10
    blk = N_NODES // grid
    return pl.pallas_call(
        _mm_body,
        grid=(grid,),
        in_specs=[
            pl.BlockSpec((blk, D), lambda i: (i, 0)),
            pl.BlockSpec((D, D), lambda i: (0, 0)),
        ],
        out_specs=pl.BlockSpec((blk, D), lambda i: (i, 0)),
        out_shape=jax.ShapeDtypeStruct((N_NODES, D), jnp.float32),
    )(x, W)


def _sc_body(h_hbm, ed_hbm, out_hbm,
             e0, e1, e2, r0, r1, r2, acc,
             i0, i1, i2, g0, g1, g2, s0, s1, s2):
    c = lax.axis_index("c")
    t = lax.axis_index("s")
    wid = c * N_SUBCORES + t
    blk0 = wid * BLOCKS_PER_TILE

    ebuf = (e0, e1, e2)
    rows = (r0, r1, r2)
    isem = (i0, i1, i2)
    gsem = (g0, g1, g2)
    ssem = (s0, s1, s2)

    @pl.loop(0, E_BLK)
    def _zero_rows(r):
        for j in range(D // 16):
            r0[r, pl.ds(16 * j, 16)] = jnp.zeros((16,), jnp.float32)

    for k, sz in ((0, 128), (128, 128), (256, 128), (384, 128), (512, 112)):
        pltpu.sync_copy(r0.at[pl.ds(0, sz)],
                        acc.at[pl.ds(t * ROWS_MAIN + k, sz)])

    @pl.when(t == N_SUBCORES - 1)
    def _zero_tail():
        pltpu.sync_copy(r0.at[pl.ds(0, ROWS_TAIL)],
                        acc.at[pl.ds(N_SUBCORES * ROWS_MAIN, ROWS_TAIL)])

    plsc.subcore_barrier()

    def idx_start(j, b):
        pltpu.async_copy(ed_hbm.at[blk0 + j], ebuf[b], isem[b])

    def idx_wait(j, b):
        pltpu.make_async_copy(ed_hbm.at[blk0 + j], ebuf[b], isem[b]).wait()

    def gather_start(j, b):
        pltpu.async_copy(h_hbm.at[ebuf[b].at[0]], rows[b], gsem[b])

    def gather_wait(j, b):
        pltpu.make_async_copy(
            h_hbm.at[ebuf[b].at[0]], rows[b], gsem[b]).wait()

    def scatter_start(j, b):
        pltpu.async_copy(rows[b], acc.at[ebuf[b].at[1]], ssem[b], add=True)

    def scatter_wait(j, b):
        pltpu.make_async_copy(
            rows[b], acc.at[ebuf[b].at[1]], ssem[b]).wait()

    def scale(b):
        buf = rows[b]
        wrow = jnp.full((16,), 2, jnp.int32)

        @pl.loop(0, E_BLK)
        def _scale(e):
            wbits = plsc.load_gather(
                ebuf[b], [wrow, jnp.full((16,), e, jnp.int32)])
            w16 = plsc.bitcast(wbits, jnp.float32)
            for u in range(D // 16):
                sl = pl.ds(16 * u, 16)
                buf[e, sl] = buf[e, sl] * w16

    idx_start(0, 0)
    idx_start(1, 1)
    idx_wait(0, 0)
    gather_start(0, 0)
    idx_wait(1, 1)
    gather_start(1, 1)

    @pl.loop(0, BLOCKS_PER_TILE, step=NBUF)
    def _edge_iter(i):
        for u in range(NBUF):
            ii = i + u
            u2 = (u + 2) % NBUF
            gather_wait(ii, u)

            @pl.when(ii >= 1)
            def _retire():
                scatter_wait(ii - 1, u2)

            @pl.when(ii + 2 < BLOCKS_PER_TILE)
            def _prefetch_idx():
                idx_start(ii + 2, u2)

            scale(u)
            scatter_start(ii, u)

            @pl.when(ii + 2 < BLOCKS_PER_TILE)
            def _refill():
                idx_wait(ii + 2, u2)
                gather_start(ii + 2, u2)

    scatter_wait(BLOCKS_PER_TILE - 1, (BLOCKS_PER_TILE - 1) % NBUF)

    plsc.subcore_barrier()

    d0 = t * ROWS_MAIN
    pltpu.sync_copy(acc.at[pl.ds(d0, ROWS_MAIN)],
                    out_hbm.at[c, pl.ds(d0, ROWS_MAIN)])

    @pl.when(t == N_SUBCORES - 1)
    def _drain_tail():
        d1 = N_SUBCORES * ROWS_MAIN
        pltpu.sync_copy(acc.at[pl.ds(d1, ROWS_TAIL)],
                        out_hbm.at[c, pl.ds(d1, ROWS_TAIL)])


def _sc_aggregate(h, edata):
    mesh = plsc.VectorSubcoreMesh(core_axis_name="c", subcore_axis_name="s")
    cp = pltpu.CompilerParams()
    if "needs_layout_passes" in pltpu.CompilerParams.__dataclass_fields__:
        cp = dataclasses.replace(cp, needs_layout_passes=False)
    kern = pl.kernel(
        _sc_body,
        out_type=jax.ShapeDtypeStruct((2, N_NODES, D), jnp.float32),
        mesh=mesh,
        scratch_types=[
            pltpu.VMEM((3, E_BLK), jnp.int32),
            pltpu.VMEM((3, E_BLK), jnp.int32),
            pltpu.VMEM((3, E_BLK), jnp.int32),
            pltpu.VMEM((E_BLK, D), jnp.float32),
            pltpu.VMEM((E_BLK, D), jnp.float32),
            pltpu.VMEM((E_BLK, D), jnp.float32),
            pltpu.VMEM_SHARED((N_NODES, D), jnp.float32),
            pltpu.SemaphoreType.DMA,
            pltpu.SemaphoreType.DMA,
            pltpu.SemaphoreType.DMA,
            pltpu.SemaphoreType.DMA,
            pltpu.SemaphoreType.DMA,
            pltpu.SemaphoreType.DMA,
            pltpu.SemaphoreType.DMA,
            pltpu.SemaphoreType.DMA,
            pltpu.SemaphoreType.DMA,
        ],
        compiler_params=cp,
    )
    return kern(h, edata)


def _add_body(p_ref, o_ref):
    o_ref[...] = p_ref[0] + p_ref[1]


def _sum_partials(partials):
    grid = 10
    blk = N_NODES // grid
    return pl.pallas_call(
        _add_body,
        grid=(grid,),
        in_specs=[pl.BlockSpec((2, blk, D), lambda i: (0, i, 0))],
        out_specs=pl.BlockSpec((blk, D), lambda i: (i, 0)),
        out_shape=jax.ShapeDtypeStruct((N_NODES, D), jnp.float32),
    )(partials)


def kernel(x, edge_index, edge_weight, W):
    h = _matmul(x, W)
    npad = E_PAD - N_EDGES
    src = jnp.concatenate(
        [edge_index[0], jnp.zeros((npad,), edge_index.dtype)])
    dst = jnp.concatenate(
        [edge_index[1], jnp.zeros((npad,), edge_index.dtype)])
    wbits = jnp.concatenate(
        [edge_weight, jnp.zeros((npad,), edge_weight.dtype)]
    ).view(jnp.int32)
    edata = jnp.stack(
        [src.reshape(N_BLOCKS, E_BLK),
         dst.reshape(N_BLOCKS, E_BLK),
         wbits.reshape(N_BLOCKS, E_BLK)], axis=1)
    partials = _sc_aggregate(h, edata)
    return _sum_partials(partials)


# speedup vs baseline: 2.7639x; 2.7639x over previous
import dataclasses
import functools

import jax
import jax.numpy as jnp
from jax import lax
from jax.experimental import pallas as pl
from jax.experimental.pallas import tpu as pltpu
from jax.experimental.pallas import tpu_sc as plsc

N_NODES = 10000
N_EDGES = 320000
D = 128

E_BLK = 128
N_BLOCKS = N_EDGES // E_BLK
BLOCKS_PER_CORE = N_BLOCKS // 2
N_SUBCORES = 16
ITERS = (BLOCKS_PER_CORE + N_SUBCORES - 1) // N_SUBCORES
ITERS_EVEN = ITERS + (ITERS % 2)
ROWS_MAIN = 624
ROWS_TAIL = N_NODES - N_SUBCORES * ROWS_MAIN


def _mm_body(x_ref, w_ref, h_ref):
    h_ref[...] = jnp.dot(x_ref[...], w_ref[...],
                         preferred_element_type=jnp.float32)


def _matmul(x, W):
    grid = 10
    blk = N_NODES // grid
    return pl.pallas_call(
        _mm_body,
        grid=(grid,),
        in_specs=[
            pl.BlockSpec((blk, D), lambda i: (i, 0)),
            pl.BlockSpec((D, D), lambda i: (0, 0)),
        ],
        out_specs=pl.BlockSpec((blk, D), lambda i: (i, 0)),
        out_shape=jax.ShapeDtypeStruct((N_NODES, D), jnp.float32),
    )(x, W)


def _sc_body(h_hbm, src_hbm, dst_hbm, w_hbm, out_hbm,
             sa, da, wa, sb, db, wb, ra, rb, acc,
             ia, ib, ga, gb):
    c = lax.axis_index("c")
    t = lax.axis_index("s")

    srcb = (sa, sb)
    dstb = (da, db)
    wbuf = (wa, wb)
    rows = (ra, rb)
    isem = (ia, ib)
    gsem = (ga, gb)

    @pl.loop(0, E_BLK)
    def _zero_rows(r):
        for j in range(D // 16):
            ra[r, pl.ds(16 * j, 16)] = jnp.zeros((16,), jnp.float32)

    for k, sz in ((0, 128), (128, 128), (256, 128), (384, 128), (512, 112)):
        pltpu.sync_copy(ra.at[pl.ds(0, sz)],
                        acc.at[pl.ds(t * ROWS_MAIN + k, sz)])

    @pl.when(t == N_SUBCORES - 1)
    def _zero_tail():
        pltpu.sync_copy(ra.at[pl.ds(0, ROWS_TAIL)],
                        acc.at[pl.ds(N_SUBCORES * ROWS_MAIN, ROWS_TAIL)])

    plsc.subcore_barrier()

    def idx_start(rel, b):
        base = (c * BLOCKS_PER_CORE + rel) * E_BLK
        pltpu.async_copy(src_hbm.at[pl.ds(base, E_BLK)], srcb[b], isem[b])
        pltpu.async_copy(dst_hbm.at[pl.ds(base, E_BLK)], dstb[b], isem[b])
        pltpu.async_copy(w_hbm.at[pl.ds(base, E_BLK)], wbuf[b], isem[b])

    def idx_wait(rel, b):
        base = (c * BLOCKS_PER_CORE + rel) * E_BLK
        pltpu.make_async_copy(
            src_hbm.at[pl.ds(base, E_BLK)], srcb[b], isem[b]).wait()
        pltpu.make_async_copy(
            dst_hbm.at[pl.ds(base, E_BLK)], dstb[b], isem[b]).wait()
        pltpu.make_async_copy(
            w_hbm.at[pl.ds(base, E_BLK)], wbuf[b], isem[b]).wait()

    def gather_start(b):
        pltpu.async_copy(h_hbm.at[srcb[b]], rows[b], gsem[b])

    def gather_wait(b):
        pltpu.make_async_copy(h_hbm.at[srcb[b]], rows[b], gsem[b]).wait()

    def scale(b):
        buf = rows[b]
        wv = wbuf[b]

        @pl.loop(0, E_BLK)
        def _scale(e):
            w16 = plsc.load_gather(wv, [jnp.full((16,), e, jnp.int32)])
            for u in range(D // 16):
                sl = pl.ds(16 * u, 16)
                buf[e, sl] = buf[e, sl] * w16

    idx_start(t, 0)
    idx_wait(t, 0)
    gather_start(0)
    idx_start(t + N_SUBCORES, 1)

    @pl.loop(0, ITERS_EVEN, step=2)
    def _edge_iter(i):
        for u in range(2):
            ii = i + u
            S = u
            O = 1 - u
            rel = ii * N_SUBCORES + t
            nrel = rel + N_SUBCORES
            nnrel = nrel + N_SUBCORES

            @pl.when(nrel < BLOCKS_PER_CORE)
            def _launch_next():
                idx_wait(nrel, O)
                gather_start(O)

            @pl.when(rel < BLOCKS_PER_CORE)
            def _process():
                gather_wait(S)
                scale(S)
                pltpu.sync_copy(rows[S], acc.at[dstb[S]], add=True)

            @pl.when(nnrel < BLOCKS_PER_CORE)
            def _prefetch_idx():
                idx_start(nnrel, S)

    plsc.subcore_barrier()

    d0 = t * ROWS_MAIN
    pltpu.sync_copy(acc.at[pl.ds(d0, ROWS_MAIN)],
                    out_hbm.at[c, pl.ds(d0, ROWS_MAIN)])

    @pl.when(t == N_SUBCORES - 1)
    def _drain_tail():
        d1 = N_SUBCORES * ROWS_MAIN
        pltpu.sync_copy(acc.at[pl.ds(d1, ROWS_TAIL)],
                        out_hbm.at[c, pl.ds(d1, ROWS_TAIL)])


def _sc_aggregate(h, src, dst, w):
    mesh = plsc.VectorSubcoreMesh(core_axis_name="c", subcore_axis_name="s")
    cp = pltpu.CompilerParams()
    if "needs_layout_passes" in pltpu.CompilerParams.__dataclass_fields__:
        cp = dataclasses.replace(cp, needs_layout_passes=False)
    kern = pl.kernel(
        _sc_body,
        out_type=jax.ShapeDtypeStruct((2, N_NODES, D), jnp.float32),
        mesh=mesh,
        scratch_types=[
            pltpu.VMEM((E_BLK,), jnp.int32),
            pltpu.VMEM((E_BLK,), jnp.int32),
            pltpu.VMEM((E_BLK,), jnp.float32),
            pltpu.VMEM((E_BLK,), jnp.int32),
            pltpu.VMEM((E_BLK,), jnp.int32),
            pltpu.VMEM((E_BLK,), jnp.float32),
            pltpu.VMEM((E_BLK, D), jnp.float32),
            pltpu.VMEM((E_BLK, D), jnp.float32),
            pltpu.VMEM_SHARED((N_NODES, D), jnp.float32),
            pltpu.SemaphoreType.DMA,
            pltpu.SemaphoreType.DMA,
            pltpu.SemaphoreType.DMA,
            pltpu.SemaphoreType.DMA,
        ],
        compiler_params=cp,
    )
    return kern(h, src, dst, w)


def _add_body(p_ref, o_ref):
    o_ref[...] = p_ref[0] + p_ref[1]


def _sum_partials(partials):
    grid = 10
    blk = N_NODES // grid
    return pl.pallas_call(
        _add_body,
        grid=(grid,),
        in_specs=[pl.BlockSpec((2, blk, D), lambda i: (0, i, 0))],
        out_specs=pl.BlockSpec((blk, D), lambda i: (i, 0)),
        out_shape=jax.ShapeDtypeStruct((N_NODES, D), jnp.float32),
    )(partials)


def kernel(x, edge_index, edge_weight, W):
    h = _matmul(x, W)
    partials = _sc_aggregate(h, edge_index[0], edge_index[1], edge_weight)
    return _sum_partials(partials)


# speedup vs baseline: 3.4145x; 1.2354x over previous
import dataclasses
import functools

import jax
import jax.numpy as jnp
from jax import lax
from jax.experimental import pallas as pl
from jax.experimental.pallas import tpu as pltpu
from jax.experimental.pallas import tpu_sc as plsc

N_NODES = 10000
N_EDGES = 320000
D = 128

E_BLK = 128
N_BLOCKS = N_EDGES // E_BLK
BLOCKS_PER_CORE = N_BLOCKS // 2
N_SUBCORES = 16
ITERS = (BLOCKS_PER_CORE + N_SUBCORES - 1) // N_SUBCORES
ITERS_PAD = 81
ROWS_MAIN = 624
ROWS_TAIL = N_NODES - N_SUBCORES * ROWS_MAIN


def _mm_body(x_ref, w_ref, h_ref):
    h_ref[...] = jnp.dot(x_ref[...], w_ref[...],
                         preferred_element_type=jnp.float32)


def _matmul(x, W):
    grid = 10
    blk = N_NODES // grid
    return pl.pallas_call(
        _mm_body,
        grid=(grid,),
        in_specs=[
            pl.BlockSpec((blk, D), lambda i: (i, 0)),
            pl.BlockSpec((D, D), lambda i: (0, 0)),
        ],
        out_specs=pl.BlockSpec((blk, D), lambda i: (i, 0)),
        out_shape=jax.ShapeDtypeStruct((N_NODES, D), jnp.float32),
    )(x, W)


def _sc_body(h_hbm, src_hbm, dst_hbm, w_hbm, out_hbm,
             sa, da, wa, sb, db, wb, sc_, dc, wc, ra, rb, rc, acc,
             ia, ib, ic, ga, gb, gc, pa, pb, pc):
    c = lax.axis_index("c")
    t = lax.axis_index("s")

    srcb = (sa, sb, sc_)
    dstb = (da, db, dc)
    wbuf = (wa, wb, wc)
    rows = (ra, rb, rc)
    isem = (ia, ib, ic)
    gsem = (ga, gb, gc)
    ssem = (pa, pb, pc)

    @pl.loop(0, E_BLK)
    def _zero_rows(r):
        for j in range(D // 16):
            ra[r, pl.ds(16 * j, 16)] = jnp.zeros((16,), jnp.float32)

    for k, sz in ((0, 128), (128, 128), (256, 128), (384, 128), (512, 112)):
        pltpu.sync_copy(ra.at[pl.ds(0, sz)],
                        acc.at[pl.ds(t * ROWS_MAIN + k, sz)])

    @pl.when(t == N_SUBCORES - 1)
    def _zero_tail():
        pltpu.sync_copy(ra.at[pl.ds(0, ROWS_TAIL)],
                        acc.at[pl.ds(N_SUBCORES * ROWS_MAIN, ROWS_TAIL)])

    plsc.subcore_barrier()

    def idx_start(rel, b):
        base = (c * BLOCKS_PER_CORE + rel) * E_BLK
        pltpu.async_copy(src_hbm.at[pl.ds(base, E_BLK)], srcb[b], isem[b])
        pltpu.async_copy(dst_hbm.at[pl.ds(base, E_BLK)], dstb[b], isem[b])
        pltpu.async_copy(w_hbm.at[pl.ds(base, E_BLK)], wbuf[b], isem[b])

    def idx_wait(rel, b):
        base = (c * BLOCKS_PER_CORE + rel) * E_BLK
        pltpu.make_async_copy(
            src_hbm.at[pl.ds(base, E_BLK)], srcb[b], isem[b]).wait()
        pltpu.make_async_copy(
            dst_hbm.at[pl.ds(base, E_BLK)], dstb[b], isem[b]).wait()
        pltpu.make_async_copy(
            w_hbm.at[pl.ds(base, E_BLK)], wbuf[b], isem[b]).wait()

    def gather_start(b):
        pltpu.async_copy(h_hbm.at[srcb[b]], rows[b], gsem[b])

    def gather_wait(b):
        pltpu.make_async_copy(h_hbm.at[srcb[b]], rows[b], gsem[b]).wait()

    def scale(b):
        buf = rows[b]
        wv = wbuf[b]

        @pl.loop(0, E_BLK)
        def _scale(e):
            w16 = plsc.load_gather(wv, [jnp.full((16,), e, jnp.int32)])
            for u in range(D // 16):
                sl = pl.ds(16 * u, 16)
                buf[e, sl] = buf[e, sl] * w16

    def scatter_start(b):
        pltpu.async_copy(rows[b], acc.at[dstb[b]], ssem[b], add=True)

    def scatter_wait(b):
        pltpu.make_async_copy(rows[b], acc.at[dstb[b]], ssem[b]).wait()

    idx_start(t, 0)
    idx_wait(t, 0)
    gather_start(0)
    idx_start(t + N_SUBCORES, 1)

    @pl.loop(0, ITERS_PAD, step=3)
    def _edge_iter(i):
        for u in range(3):
            k = i + u
            S = u
            P = (u + 1) % 3
            Q = (u + 2) % 3
            rel = k * N_SUBCORES + t

            @pl.when(rel + N_SUBCORES < BLOCKS_PER_CORE)
            def _launch_next():
                idx_wait(rel + N_SUBCORES, P)
                gather_start(P)

            @pl.when(rel < BLOCKS_PER_CORE)
            def _process():
                gather_wait(S)
                scale(S)
                scatter_start(S)

            @pl.when((k >= 1) & (rel - N_SUBCORES < BLOCKS_PER_CORE))
            def _retire_prev():
                scatter_wait(Q)

            @pl.when(rel + 2 * N_SUBCORES < BLOCKS_PER_CORE)
            def _prefetch_idx():
                idx_start(rel + 2 * N_SUBCORES, Q)

    plsc.subcore_barrier()

    d0 = t * ROWS_MAIN
    pltpu.sync_copy(acc.at[pl.ds(d0, ROWS_MAIN)],
                    out_hbm.at[c, pl.ds(d0, ROWS_MAIN)])

    @pl.when(t == N_SUBCORES - 1)
    def _drain_tail():
        d1 = N_SUBCORES * ROWS_MAIN
        pltpu.sync_copy(acc.at[pl.ds(d1, ROWS_TAIL)],
                        out_hbm.at[c, pl.ds(d1, ROWS_TAIL)])


def _sc_aggregate(h, src, dst, w):
    mesh = plsc.VectorSubcoreMesh(core_axis_name="c", subcore_axis_name="s")
    cp = pltpu.CompilerParams()
    if "needs_layout_passes" in pltpu.CompilerParams.__dataclass_fields__:
        cp = dataclasses.replace(cp, needs_layout_passes=False)
    kern = pl.kernel(
        _sc_body,
        out_type=jax.ShapeDtypeStruct((2, N_NODES, D), jnp.float32),
        mesh=mesh,
        scratch_types=[
            pltpu.VMEM((E_BLK,), jnp.int32),
            pltpu.VMEM((E_BLK,), jnp.int32),
            pltpu.VMEM((E_BLK,), jnp.float32),
            pltpu.VMEM((E_BLK,), jnp.int32),
            pltpu.VMEM((E_BLK,), jnp.int32),
            pltpu.VMEM((E_BLK,), jnp.float32),
            pltpu.VMEM((E_BLK,), jnp.int32),
            pltpu.VMEM((E_BLK,), jnp.int32),
            pltpu.VMEM((E_BLK,), jnp.float32),
            pltpu.VMEM((E_BLK, D), jnp.float32),
            pltpu.VMEM((E_BLK, D), jnp.float32),
            pltpu.VMEM((E_BLK, D), jnp.float32),
            pltpu.VMEM_SHARED((N_NODES, D), jnp.float32),
            pltpu.SemaphoreType.DMA,
            pltpu.SemaphoreType.DMA,
            pltpu.SemaphoreType.DMA,
            pltpu.SemaphoreType.DMA,
            pltpu.SemaphoreType.DMA,
            pltpu.SemaphoreType.DMA,
            pltpu.SemaphoreType.DMA,
            pltpu.SemaphoreType.DMA,
            pltpu.SemaphoreType.DMA,
        ],
        compiler_params=cp,
    )
    return kern(h, src, dst, w)


def _add_body(p_ref, o_ref):
    o_ref[...] = p_ref[0] + p_ref[1]


def _sum_partials(partials):
    grid = 10
    blk = N_NODES // grid
    return pl.pallas_call(
        _add_body,
        grid=(grid,),
        in_specs=[pl.BlockSpec((2, blk, D), lambda i: (0, i, 0))],
        out_specs=pl.BlockSpec((blk, D), lambda i: (i, 0)),
        out_shape=jax.ShapeDtypeStruct((N_NODES, D), jnp.float32),
    )(partials)


def kernel(x, edge_index, edge_weight, W):
    h = _matmul(x, W)
    partials = _sc_aggregate(h, edge_index[0], edge_index[1], edge_weight)
    return _sum_partials(partials)


# speedup vs baseline: 3.5846x; 1.0498x over previous
import dataclasses
import functools

import jax
import jax.numpy as jnp
from jax import lax
from jax.experimental import pallas as pl
from jax.experimental.pallas import tpu as pltpu
from jax.experimental.pallas import tpu_sc as plsc

N_NODES = 10000
N_EDGES = 320000
D = 128

E_BLK = 128
N_BLOCKS = N_EDGES // E_BLK
BLOCKS_PER_CORE = N_BLOCKS // 2
N_SUBCORES = 16
ITERS = (BLOCKS_PER_CORE + N_SUBCORES - 1) // N_SUBCORES
ITERS_PAD = 81
ROWS_MAIN = 624
ROWS_TAIL = N_NODES - N_SUBCORES * ROWS_MAIN


def _mm_body(x_ref, w_ref, h_ref):
    h_ref[...] = jnp.dot(x_ref[...], w_ref[...],
                         preferred_element_type=jnp.float32)


def _matmul(x, W):
    grid = 10
    blk = N_NODES // grid
    return pl.pallas_call(
        _mm_body,
        grid=(grid,),
        in_specs=[
            pl.BlockSpec((blk, D), lambda i: (i, 0)),
            pl.BlockSpec((D, D), lambda i: (0, 0)),
        ],
        out_specs=pl.BlockSpec((blk, D), lambda i: (i, 0)),
        out_shape=jax.ShapeDtypeStruct((N_NODES, D), jnp.float32),
    )(x, W)


def _sc_body(h_hbm, src_hbm, dst_hbm, w_hbm, out_hbm,
             sa, da, wa, sb, db, wb, sc_, dc, wc, ra, rb, rc, acc,
             ia, ib, ic, ga, gb, gc, pa, pb, pc):
    c = lax.axis_index("c")
    t = lax.axis_index("s")

    srcb = (sa, sb, sc_)
    dstb = (da, db, dc)
    wbuf = (wa, wb, wc)
    rows = (ra, rb, rc)
    isem = (ia, ib, ic)
    gsem = (ga, gb, gc)
    ssem = (pa, pb, pc)

    @pl.loop(0, E_BLK)
    def _zero_rows(r):
        for j in range(D // 16):
            ra[r, pl.ds(16 * j, 16)] = jnp.zeros((16,), jnp.float32)

    for k, sz in ((0, 128), (128, 128), (256, 128), (384, 128), (512, 112)):
        pltpu.sync_copy(ra.at[pl.ds(0, sz)],
                        acc.at[pl.ds(t * ROWS_MAIN + k, sz)])

    @pl.when(t == N_SUBCORES - 1)
    def _zero_tail():
        pltpu.sync_copy(ra.at[pl.ds(0, ROWS_TAIL)],
                        acc.at[pl.ds(N_SUBCORES * ROWS_MAIN, ROWS_TAIL)])

    plsc.subcore_barrier()

    def idx_start(rel, b):
        base = (c * BLOCKS_PER_CORE + rel) * E_BLK
        pltpu.async_copy(src_hbm.at[pl.ds(base, E_BLK)], srcb[b], isem[b])
        pltpu.async_copy(dst_hbm.at[pl.ds(base, E_BLK)], dstb[b], isem[b])
        pltpu.async_copy(w_hbm.at[pl.ds(base, E_BLK)], wbuf[b], isem[b])

    def idx_wait(rel, b):
        base = (c * BLOCKS_PER_CORE + rel) * E_BLK
        pltpu.make_async_copy(
            src_hbm.at[pl.ds(base, E_BLK)], srcb[b], isem[b]).wait()
        pltpu.make_async_copy(
            dst_hbm.at[pl.ds(base, E_BLK)], dstb[b], isem[b]).wait()
        pltpu.make_async_copy(
            w_hbm.at[pl.ds(base, E_BLK)], wbuf[b], isem[b]).wait()

    def gather_start(b):
        pltpu.async_copy(h_hbm.at[srcb[b]], rows[b], gsem[b])

    def gather_wait(b):
        pltpu.make_async_copy(h_hbm.at[srcb[b]], rows[b], gsem[b]).wait()

    def scale(b):
        buf = rows[b]
        wv = wbuf[b]

        @pl.loop(0, E_BLK, step=4)
        def _scale(e):
            for v in range(4):
                w16 = plsc.load_gather(wv, [jnp.full((16,), e + v, jnp.int32)])
                for u in range(D // 16):
                    sl = pl.ds(16 * u, 16)
                    buf[e + v, sl] = buf[e + v, sl] * w16

    def scatter_start(b):
        pltpu.async_copy(rows[b], acc.at[dstb[b]], ssem[b], add=True)

    def scatter_wait(b):
        pltpu.make_async_copy(rows[b], acc.at[dstb[b]], ssem[b]).wait()

    idx_start(t, 0)
    idx_wait(t, 0)
    gather_start(0)
    idx_start(t + N_SUBCORES, 1)

    @pl.loop(0, ITERS_PAD, step=3)
    def _edge_iter(i):
        for u in range(3):
            k = i + u
            S = u
            P = (u + 1) % 3
            Q = (u + 2) % 3
            rel = k * N_SUBCORES + t

            @pl.when(rel + N_SUBCORES < BLOCKS_PER_CORE)
            def _launch_next():
                idx_wait(rel + N_SUBCORES, P)
                gather_start(P)

            @pl.when(rel < BLOCKS_PER_CORE)
            def _process():
                gather_wait(S)
                scale(S)
                scatter_start(S)

            @pl.when((k >= 1) & (rel - N_SUBCORES < BLOCKS_PER_CORE))
            def _retire_prev():
                scatter_wait(Q)

            @pl.when(rel + 2 * N_SUBCORES < BLOCKS_PER_CORE)
            def _prefetch_idx():
                idx_start(rel + 2 * N_SUBCORES, Q)

    plsc.subcore_barrier()

    d0 = t * ROWS_MAIN
    pltpu.sync_copy(acc.at[pl.ds(d0, ROWS_MAIN)],
                    out_hbm.at[c, pl.ds(d0, ROWS_MAIN)])

    @pl.when(t == N_SUBCORES - 1)
    def _drain_tail():
        d1 = N_SUBCORES * ROWS_MAIN
        pltpu.sync_copy(acc.at[pl.ds(d1, ROWS_TAIL)],
                        out_hbm.at[c, pl.ds(d1, ROWS_TAIL)])


def _sc_aggregate(h, src, dst, w):
    mesh = plsc.VectorSubcoreMesh(core_axis_name="c", subcore_axis_name="s")
    cp = pltpu.CompilerParams()
    if "needs_layout_passes" in pltpu.CompilerParams.__dataclass_fields__:
        cp = dataclasses.replace(cp, needs_layout_passes=False)
    kern = pl.kernel(
        _sc_body,
        out_type=jax.ShapeDtypeStruct((2, N_NODES, D), jnp.float32),
        mesh=mesh,
        scratch_types=[
            pltpu.VMEM((E_BLK,), jnp.int32),
            pltpu.VMEM((E_BLK,), jnp.int32),
            pltpu.VMEM((E_BLK,), jnp.float32),
            pltpu.VMEM((E_BLK,), jnp.int32),
            pltpu.VMEM((E_BLK,), jnp.int32),
            pltpu.VMEM((E_BLK,), jnp.float32),
            pltpu.VMEM((E_BLK,), jnp.int32),
            pltpu.VMEM((E_BLK,), jnp.int32),
            pltpu.VMEM((E_BLK,), jnp.float32),
            pltpu.VMEM((E_BLK, D), jnp.float32),
            pltpu.VMEM((E_BLK, D), jnp.float32),
            pltpu.VMEM((E_BLK, D), jnp.float32),
            pltpu.VMEM_SHARED((N_NODES, D), jnp.float32),
            pltpu.SemaphoreType.DMA,
            pltpu.SemaphoreType.DMA,
            pltpu.SemaphoreType.DMA,
            pltpu.SemaphoreType.DMA,
            pltpu.SemaphoreType.DMA,
            pltpu.SemaphoreType.DMA,
            pltpu.SemaphoreType.DMA,
            pltpu.SemaphoreType.DMA,
            pltpu.SemaphoreType.DMA,
        ],
        compiler_params=cp,
    )
    return kern(h, src, dst, w)


def _add_body(p_ref, o_ref):
    o_ref[...] = p_ref[0] + p_ref[1]


def _sum_partials(partials):
    grid = 10
    blk = N_NODES // grid
    return pl.pallas_call(
        _add_body,
        grid=(grid,),
        in_specs=[pl.BlockSpec((2, blk, D), lambda i: (0, i, 0))],
        out_specs=pl.BlockSpec((blk, D), lambda i: (i, 0)),
        out_shape=jax.ShapeDtypeStruct((N_NODES, D), jnp.float32),
    )(partials)


def kernel(x, edge_index, edge_weight, W):
    h = _matmul(x, W)
    partials = _sc_aggregate(h, edge_index[0], edge_index[1], edge_weight)
    return _sum_partials(partials)


# speedup vs baseline: 3.6345x; 1.0139x over previous
import dataclasses
import functools

import jax
import jax.numpy as jnp
from jax import lax
from jax.experimental import pallas as pl
from jax.experimental.pallas import tpu as pltpu
from jax.experimental.pallas import tpu_sc as plsc

N_NODES = 10000
N_EDGES = 320000
D = 128

E_BLK = 128
N_BLOCKS = N_EDGES // E_BLK
BLOCKS_PER_CORE = N_BLOCKS // 2
N_SUBCORES = 16
ITERS = (BLOCKS_PER_CORE + N_SUBCORES - 1) // N_SUBCORES
ITERS_PAD = 81
ROWS_MAIN = 624
ROWS_TAIL = N_NODES - N_SUBCORES * ROWS_MAIN


def _mm_body(x_ref, w_ref, h_ref):
    h_ref[...] = jnp.dot(x_ref[...], w_ref[...],
                         preferred_element_type=jnp.float32)


def _matmul(x, W):
    grid = 10
    blk = N_NODES // grid
    return pl.pallas_call(
        _mm_body,
        grid=(grid,),
        in_specs=[
            pl.BlockSpec((blk, D), lambda i: (i, 0)),
            pl.BlockSpec((D, D), lambda i: (0, 0)),
        ],
        out_specs=pl.BlockSpec((blk, D), lambda i: (i, 0)),
        out_shape=jax.ShapeDtypeStruct((N_NODES, D), jnp.float32),
    )(x, W)


def _sc_body(h_hbm, meta_hbm, out_hbm,
             ma, mb_, mc, ra, rb, rc, acc,
             ia, ib, ic, ga, gb, gc, pa, pb, pc):
    c = lax.axis_index("c")
    t = lax.axis_index("s")

    meta = (ma, mb_, mc)
    rows = (ra, rb, rc)
    isem = (ia, ib, ic)
    gsem = (ga, gb, gc)
    ssem = (pa, pb, pc)

    @pl.loop(0, E_BLK)
    def _zero_rows(r):
        for j in range(D // 16):
            ra[r, pl.ds(16 * j, 16)] = jnp.zeros((16,), jnp.float32)

    for k, sz in ((0, 128), (128, 128), (256, 128), (384, 128), (512, 112)):
        pltpu.sync_copy(ra.at[pl.ds(0, sz)],
                        acc.at[pl.ds(t * ROWS_MAIN + k, sz)])

    @pl.when(t == N_SUBCORES - 1)
    def _zero_tail():
        pltpu.sync_copy(ra.at[pl.ds(0, ROWS_TAIL)],
                        acc.at[pl.ds(N_SUBCORES * ROWS_MAIN, ROWS_TAIL)])

    plsc.subcore_barrier()

    def idx_start(rel, b):
        base = c * BLOCKS_PER_CORE + rel
        pltpu.async_copy(meta_hbm.at[base], meta[b], isem[b])

    def idx_wait(rel, b):
        base = c * BLOCKS_PER_CORE + rel
        pltpu.make_async_copy(meta_hbm.at[base], meta[b], isem[b]).wait()

    def gather_start(b):
        pltpu.async_copy(h_hbm.at[meta[b].at[0]], rows[b], gsem[b])

    def gather_wait(b):
        pltpu.make_async_copy(h_hbm.at[meta[b].at[0]], rows[b],
                              gsem[b]).wait()

    def scale(b):
        buf = rows[b]
        wref = meta[b]

        @pl.loop(0, E_BLK, step=4)
        def _scale(e):
            for v in range(4):
                w16i = plsc.load_gather(
                    wref, [jnp.full((16,), 2, jnp.int32),
                           jnp.full((16,), e + v, jnp.int32)])
                w16 = plsc.bitcast(w16i, jnp.float32)
                for u in range(D // 16):
                    sl = pl.ds(16 * u, 16)
                    buf[e + v, sl] = buf[e + v, sl] * w16

    def scatter_start(b):
        pltpu.async_copy(rows[b], acc.at[meta[b].at[1]], ssem[b], add=True)

    def scatter_wait(b):
        pltpu.make_async_copy(rows[b], acc.at[meta[b].at[1]],
                              ssem[b]).wait()

    idx_start(t, 0)
    idx_wait(t, 0)
    gather_start(0)
    idx_start(t + N_SUBCORES, 1)

    @pl.loop(0, ITERS_PAD, step=3)
    def _edge_iter(i):
        for u in range(3):
            k = i + u
            S = u
            P = (u + 1) % 3
            Q = (u + 2) % 3
            rel = k * N_SUBCORES + t

            @pl.when(rel + N_SUBCORES < BLOCKS_PER_CORE)
            def _launch_next():
                idx_wait(rel + N_SUBCORES, P)
                gather_start(P)

            @pl.when(rel < BLOCKS_PER_CORE)
            def _process():
                gather_wait(S)
                scale(S)
                scatter_start(S)

            @pl.when((k >= 1) & (rel - N_SUBCORES < BLOCKS_PER_CORE))
            def _retire_prev():
                scatter_wait(Q)

            @pl.when(rel + 2 * N_SUBCORES < BLOCKS_PER_CORE)
            def _prefetch_idx():
                idx_start(rel + 2 * N_SUBCORES, Q)

    plsc.subcore_barrier()

    d0 = t * ROWS_MAIN
    pltpu.sync_copy(acc.at[pl.ds(d0, ROWS_MAIN)],
                    out_hbm.at[c, pl.ds(d0, ROWS_MAIN)])

    @pl.when(t == N_SUBCORES - 1)
    def _drain_tail():
        d1 = N_SUBCORES * ROWS_MAIN
        pltpu.sync_copy(acc.at[pl.ds(d1, ROWS_TAIL)],
                        out_hbm.at[c, pl.ds(d1, ROWS_TAIL)])


def _sc_aggregate(h, meta):
    mesh = plsc.VectorSubcoreMesh(core_axis_name="c", subcore_axis_name="s")
    cp = pltpu.CompilerParams()
    if "needs_layout_passes" in pltpu.CompilerParams.__dataclass_fields__:
        cp = dataclasses.replace(cp, needs_layout_passes=False)
    kern = pl.kernel(
        _sc_body,
        out_type=jax.ShapeDtypeStruct((2, N_NODES, D), jnp.float32),
        mesh=mesh,
        scratch_types=[
            pltpu.VMEM((3, E_BLK), jnp.int32),
            pltpu.VMEM((3, E_BLK), jnp.int32),
            pltpu.VMEM((3, E_BLK), jnp.int32),
            pltpu.VMEM((E_BLK, D), jnp.float32),
            pltpu.VMEM((E_BLK, D), jnp.float32),
            pltpu.VMEM((E_BLK, D), jnp.float32),
            pltpu.VMEM_SHARED((N_NODES, D), jnp.float32),
            pltpu.SemaphoreType.DMA,
            pltpu.SemaphoreType.DMA,
            pltpu.SemaphoreType.DMA,
            pltpu.SemaphoreType.DMA,
            pltpu.SemaphoreType.DMA,
            pltpu.SemaphoreType.DMA,
            pltpu.SemaphoreType.DMA,
            pltpu.SemaphoreType.DMA,
            pltpu.SemaphoreType.DMA,
        ],
        compiler_params=cp,
    )
    return kern(h, meta)


def _add_body(p_ref, o_ref):
    o_ref[...] = p_ref[0] + p_ref[1]


def _sum_partials(partials):
    grid = 10
    blk = N_NODES // grid
    return pl.pallas_call(
        _add_body,
        grid=(grid,),
        in_specs=[pl.BlockSpec((2, blk, D), lambda i: (0, i, 0))],
        out_specs=pl.BlockSpec((blk, D), lambda i: (i, 0)),
        out_shape=jax.ShapeDtypeStruct((N_NODES, D), jnp.float32),
    )(partials)


def kernel(x, edge_index, edge_weight, W):
    h = _matmul(x, W)
    srcr = edge_index[0].reshape(N_BLOCKS, 1, E_BLK)
    dstr = edge_index[1].reshape(N_BLOCKS, 1, E_BLK)
    wbits = lax.bitcast_convert_type(edge_weight, jnp.int32)
    wr = wbits.reshape(N_BLOCKS, 1, E_BLK)
    meta = jnp.concatenate([srcr, dstr, wr], axis=1)
    partials = _sc_aggregate(h, meta)
    return _sum_partials(partials)
